# baseline (speedup 1.0000x reference)
import functools

import jax
import jax.numpy as jnp
from jax import lax
from jax.experimental import pallas as pl
from jax.experimental.pallas import tpu as pltpu
from jax.experimental.pallas import tpu_sc as plsc

_D = 128
_BATCH = 16384
_HIST = 50
_NC, _NS = 2, 16
_NW = _NC * _NS
_P = 4
_BSLICE = _BATCH // _P
_BPW = _BSLICE // _NW
_G = 2
_CHUNKS = _BPW // _G
_NBUF = 3
_MAIN = -(-(_CHUNKS - _NBUF) // _NBUF) * _NBUF

_mesh = plsc.VectorSubcoreMesh(core_axis_name="c", subcore_axis_name="s")


@functools.partial(
    pl.kernel,
    out_type=jax.ShapeDtypeStruct((_BSLICE, _HIST, _D), jnp.float32),
    mesh=_mesh,
    scratch_types=[
        pltpu.VMEM((_BPW, _HIST), jnp.int32),
        pltpu.VMEM((_NBUF, _G, _HIST, _D), jnp.float32),
        pltpu.SemaphoreType.DMA,
        pltpu.SemaphoreType.DMA,
        pltpu.SemaphoreType.DMA,
    ],
)
def _emb(x_hbm, w_hbm, out_hbm, idx_v, rows_v, sem0, sem1, sem2):
    wid = lax.axis_index("s") * _NC + lax.axis_index("c")
    base = wid * _BPW
    sems = (sem0, sem1, sem2)
    pltpu.sync_copy(x_hbm.at[pl.ds(base, _BPW)], idx_v)

    def _fire(c, buf):
        r0 = c * _G
        for j in range(_G):
            pltpu.async_copy(
                w_hbm.at[idx_v.at[r0 + j]],
                rows_v.at[buf].at[j],
                sems[buf],
            )

    def _drain_store(c, b):
        for j in range(_G):
            pltpu.make_async_copy(
                out_hbm.at[0], rows_v.at[b].at[j], sems[b]
            ).wait()
        pltpu.sync_copy(
            rows_v.at[b],
            out_hbm.at[pl.ds(base + c * _G, _G)],
        )

    for b in range(_NBUF):
        _fire(b, b)

    @pl.loop(0, _MAIN, step=_NBUF)
    def _outer(cc):
        for b in range(_NBUF):
            c = cc + b
            _drain_store(c, b)

            @pl.when(c + _NBUF < _CHUNKS)
            def _():
                _fire(c + _NBUF, b)

    for c in range(_MAIN, _CHUNKS):
        _drain_store(c, c % _NBUF)


def kernel(x, weight):
    xi = x.astype(jnp.int32)
    outs = [
        _emb(lax.slice_in_dim(xi, i * _BSLICE, (i + 1) * _BSLICE), weight)
        for i in range(_P)
    ]
    out = jnp.zeros((_BATCH, _HIST, _D), jnp.float32)
    for i, o in enumerate(outs):
        out = lax.optimization_barrier(
            lax.dynamic_update_slice(out, o, (i * _BSLICE, 0, 0))
        )
    return out


# speedup vs baseline: 1.6113x; 1.6113x over previous
import functools

import jax
import jax.numpy as jnp
from jax import lax
from jax.experimental import pallas as pl
from jax.experimental.pallas import tpu as pltpu
from jax.experimental.pallas import tpu_sc as plsc

_D = 128
_BATCH = 16384
_HIST = 50
_NC, _NS = 2, 16
_NW = _NC * _NS
_P = 1
_BSLICE = _BATCH // _P
_BPW = _BSLICE // _NW
_G = 2
_CHUNKS = _BPW // _G
_NBUF = 3
_MAIN = -(-(_CHUNKS - _NBUF) // _NBUF) * _NBUF

_mesh = plsc.VectorSubcoreMesh(core_axis_name="c", subcore_axis_name="s")


@functools.partial(
    pl.kernel,
    out_type=jax.ShapeDtypeStruct((_BSLICE, _HIST, _D), jnp.float32),
    mesh=_mesh,
    compiler_params=pltpu.CompilerParams(needs_layout_passes=True),
    scratch_types=[
        pltpu.VMEM((_BPW, _HIST), jnp.int32),
        pltpu.VMEM((_NBUF, _G, _HIST, _D), jnp.float32),
        pltpu.SemaphoreType.DMA,
        pltpu.SemaphoreType.DMA,
        pltpu.SemaphoreType.DMA,
    ],
)
def _emb(x_hbm, w_hbm, out_hbm, idx_v, rows_v, sem0, sem1, sem2):
    wid = lax.axis_index("s") * _NC + lax.axis_index("c")
    base = wid * _BPW
    sems = (sem0, sem1, sem2)
    pltpu.sync_copy(x_hbm.at[pl.ds(base, _BPW)], idx_v)

    def _fire(c, buf):
        r0 = c * _G
        for j in range(_G):
            pltpu.async_copy(
                w_hbm.at[idx_v.at[r0 + j]],
                rows_v.at[buf].at[j],
                sems[buf],
            )

    def _drain_store(c, b):
        for j in range(_G):
            pltpu.make_async_copy(
                out_hbm.at[0], rows_v.at[b].at[j], sems[b]
            ).wait()
        pltpu.sync_copy(
            rows_v.at[b],
            out_hbm.at[pl.ds(base + c * _G, _G)],
        )

    for b in range(_NBUF):
        _fire(b, b)

    @pl.loop(0, _MAIN, step=_NBUF)
    def _outer(cc):
        for b in range(_NBUF):
            c = cc + b
            _drain_store(c, b)

            @pl.when(c + _NBUF < _CHUNKS)
            def _():
                _fire(c + _NBUF, b)

    for c in range(_MAIN, _CHUNKS):
        _drain_store(c, c % _NBUF)


def kernel(x, weight):
    xi = x.astype(jnp.int32)
    outs = [
        _emb(lax.slice_in_dim(xi, i * _BSLICE, (i + 1) * _BSLICE), weight)
        for i in range(_P)
    ]
    out = jnp.zeros((_BATCH, _HIST, _D), jnp.float32)
    for i, o in enumerate(outs):
        out = lax.optimization_barrier(
            lax.dynamic_update_slice(out, o, (i * _BSLICE, 0, 0))
        )
    return out


# speedup vs baseline: 1.8226x; 1.1311x over previous
import functools

import jax
import jax.numpy as jnp
from jax import lax
from jax.experimental import pallas as pl
from jax.experimental.pallas import tpu as pltpu
from jax.experimental.pallas import tpu_sc as plsc

_D = 128
_BATCH = 16384
_HIST = 50
_NC, _NS = 2, 16
_NW = _NC * _NS
_P = 1
_BSLICE = _BATCH // _P
_BPW = _BSLICE // _NW
_G = 2
_CHUNKS = _BPW // _G
_NBUF = 3
_MAIN = -(-(_CHUNKS - _NBUF) // _NBUF) * _NBUF

_mesh = plsc.VectorSubcoreMesh(core_axis_name="c", subcore_axis_name="s")


@functools.partial(
    pl.kernel,
    out_type=jax.ShapeDtypeStruct((_BSLICE, _HIST, _D), jnp.float32),
    mesh=_mesh,
    compiler_params=pltpu.CompilerParams(needs_layout_passes=True),
    scratch_types=[
        pltpu.VMEM((_BPW, _HIST), jnp.int32),
        pltpu.VMEM((_NBUF, _G, _HIST, _D), jnp.float32),
        pltpu.SemaphoreType.DMA,
        pltpu.SemaphoreType.DMA,
        pltpu.SemaphoreType.DMA,
    ],
)
def _emb(x_hbm, w_hbm, out_hbm, idx_v, rows_v, sem0, sem1, sem2):
    wid = lax.axis_index("s") * _NC + lax.axis_index("c")
    base = wid * _BPW
    sems = (sem0, sem1, sem2)
    pltpu.sync_copy(x_hbm.at[pl.ds(base, _BPW)], idx_v)

    def _fire(c, buf):
        r0 = c * _G
        for j in range(_G):
            pltpu.async_copy(
                w_hbm.at[idx_v.at[r0 + j]],
                rows_v.at[buf].at[j],
                sems[buf],
            )

    def _drain_store(c, b):
        for j in range(_G):
            pltpu.make_async_copy(
                out_hbm.at[0], rows_v.at[b].at[j], sems[b]
            ).wait()
        pltpu.sync_copy(
            rows_v.at[b],
            out_hbm.at[pl.ds(base + c * _G, _G)],
        )

    for b in range(_NBUF):
        _fire(b, b)

    @pl.loop(0, _MAIN, step=_NBUF)
    def _outer(cc):
        for b in range(_NBUF):
            c = cc + b
            _drain_store(c, b)

            @pl.when(c + _NBUF < _CHUNKS)
            def _():
                _fire(c + _NBUF, b)

    for c in range(_MAIN, _CHUNKS):
        _drain_store(c, c % _NBUF)


def kernel(x, weight):
    xi = x.astype(jnp.int32)
    outs = [
        _emb(lax.slice_in_dim(xi, i * _BSLICE, (i + 1) * _BSLICE), weight)
        for i in range(_P)
    ]
    if _P == 1:
        return outs[0]
    out = jnp.zeros((_BATCH, _HIST, _D), jnp.float32)
    for i, o in enumerate(outs):
        out = lax.optimization_barrier(
            lax.dynamic_update_slice(out, o, (i * _BSLICE, 0, 0))
        )
    return out


# speedup vs baseline: 1.8229x; 1.0002x over previous
import functools

import jax
import jax.numpy as jnp
from jax import lax
from jax.experimental import pallas as pl
from jax.experimental.pallas import tpu as pltpu
from jax.experimental.pallas import tpu_sc as plsc

_D = 128
_BATCH = 16384
_HIST = 50
_NC, _NS = 2, 16
_NW = _NC * _NS
_P = 1
_BSLICE = _BATCH // _P
_BPW = _BSLICE // _NW
_G = 4
_CHUNKS = _BPW // _G
_NBUF = 2
_MAIN = -(-(_CHUNKS - _NBUF) // _NBUF) * _NBUF

_mesh = plsc.VectorSubcoreMesh(core_axis_name="c", subcore_axis_name="s")


@functools.partial(
    pl.kernel,
    out_type=jax.ShapeDtypeStruct((_BSLICE, _HIST, _D), jnp.float32),
    mesh=_mesh,
    scratch_types=[
        pltpu.VMEM((_BPW, _HIST), jnp.int32),
        pltpu.VMEM((_NBUF, _G, _HIST, _D), jnp.float32),
        pltpu.SemaphoreType.DMA,
        pltpu.SemaphoreType.DMA,
    ],
)
def _emb(x_hbm, w_hbm, out_hbm, idx_v, rows_v, sem0, sem1):
    wid = lax.axis_index("s") * _NC + lax.axis_index("c")
    base = wid * _BPW
    sems = (sem0, sem1)
    pltpu.sync_copy(x_hbm.at[pl.ds(base, _BPW)], idx_v)

    def _fire(c, buf):
        r0 = c * _G
        for j in range(_G):
            pltpu.async_copy(
                w_hbm.at[idx_v.at[r0 + j]],
                rows_v.at[buf].at[j],
                sems[buf],
            )

    def _drain_store(c, b):
        pltpu.make_async_copy(
            out_hbm.at[pl.ds(0, _G)], rows_v.at[b], sems[b]
        ).wait()
        pltpu.sync_copy(
            rows_v.at[b],
            out_hbm.at[pl.ds(base + c * _G, _G)],
        )

    for b in range(_NBUF):
        _fire(b, b)

    @pl.loop(0, _MAIN, step=_NBUF)
    def _outer(cc):
        for b in range(_NBUF):
            c = cc + b
            _drain_store(c, b)

            @pl.when(c + _NBUF < _CHUNKS)
            def _():
                _fire(c + _NBUF, b)

    for c in range(_MAIN, _CHUNKS):
        _drain_store(c, c % _NBUF)


def kernel(x, weight):
    xi = x.astype(jnp.int32)
    outs = [
        _emb(lax.slice_in_dim(xi, i * _BSLICE, (i + 1) * _BSLICE), weight)
        for i in range(_P)
    ]
    if _P == 1:
        return outs[0]
    out = jnp.zeros((_BATCH, _HIST, _D), jnp.float32)
    for i, o in enumerate(outs):
        out = lax.optimization_barrier(
            lax.dynamic_update_slice(out, o, (i * _BSLICE, 0, 0))
        )
    return out
